# baseline (speedup 1.0000x reference)
import functools

import numpy as np
import jax
import jax.numpy as jnp
from jax import lax
from jax.experimental import pallas as pl
from jax.experimental.pallas import tpu as pltpu
from jax.experimental.pallas import tpu_sc as plsc

_J = 4
_L = 4
_M = 128
_N = 128


def _build_tables():
    ii = np.arange(_M)
    sx = ((ii + _M // 2) % _M) - _M // 2
    sy = ((np.arange(_N) + _N // 2) % _N) - _N // 2
    SX, SY = np.meshgrid(sx, sy, indexing="ij")
    r = np.sqrt(SX.astype(np.float64) ** 2 + SY.astype(np.float64) ** 2)
    angle_ok = (SX == 0) | (SY == 0) | (SX == SY) | (SX == -SY)
    masks = [(SX == 0) & (SY == 0)]
    for k in range(1, _J + 1):
        masks.append((r <= 2 ** (k - 1)) & angle_ok)
    positions = [np.where(m.reshape(-1))[0] for m in masks]

    la1, la2, kk = [], [], []
    for j1 in range(_J):
        for j2 in range(j1, _J):
            for l1 in range(_L):
                for l2 in range(_L):
                    la1.append(_L * j1 + l1)
                    la2.append(_L * j2 + l2)
                    kk.append(j2 + 1)

    all_shifts = set()
    for q in positions[_J]:
        i, j = divmod(int(q), _N)
        all_shifts.add((int(sx[i]), int(sy[j])))
    groups = {}
    for dx, dy in sorted(all_shifts):
        groups.setdefault(dx, []).append(dy)
    shift_groups = sorted(groups.items(),
                          key=lambda it: ((it[0] % _M) % 8, it[0] % _M))
    sidx_of = {}
    s = 0
    for dx, dys in shift_groups:
        for dy in dys:
            sidx_of[(dx, dy)] = s
            s += 1

    idx = []
    for p in range(len(la1)):
        for q in positions[kk[p]]:
            i, j = divmod(int(q), _N)
            si = sidx_of[(int(sx[i]), int(sy[j]))]
            idx.append((si, la1[p], la2[p]))
    return shift_groups, np.asarray(idx, dtype=np.int32)


_SHIFT_GROUPS, _IDX3 = _build_tables()
_NS = sum(len(d) for _, d in _SHIFT_GROUPS)
_P_OUT = int(_IDX3.shape[0])
_CHUNK = _P_OUT // 4
_CPAD = 1280

_IDX_S = np.zeros((4 * _CPAD,), dtype=np.int32)
_IDX_R = np.zeros((4 * _CPAD,), dtype=np.int32)
_IDX_C = np.zeros((4 * _CPAD,), dtype=np.int32)
for _sub in range(4):
    _part = _IDX3[_sub * _CHUNK:(_sub + 1) * _CHUNK]
    _IDX_S[_sub * _CPAD:_sub * _CPAD + _CHUNK] = _part[:, 0]
    _IDX_R[_sub * _CPAD:_sub * _CPAD + _CHUNK] = _part[:, 1]
    _IDX_C[_sub * _CPAD:_sub * _CPAD + _CHUNK] = _part[:, 2]


def _flat_roll(a, k):
    k %= _M * _N
    if k == 0:
        return a
    return jnp.concatenate([a[:, k:], a[:, :k]], axis=1)


def _gram_body(x_ref, out_ref):
    a = x_ref[0]
    lane = jax.lax.broadcasted_iota(jnp.int32, (16, _M * _N), 1) % _N
    s = 0
    for dx, dys in _SHIFT_GROUPS:
        for dy in dys:
            dym = dy % _N
            ka = _N * dx + dym
            if dym == 0:
                xs = _flat_roll(a, ka)
            else:
                xs = jnp.where(lane < _N - dym,
                               _flat_roll(a, ka), _flat_roll(a, ka - _N))
            g = lax.dot_general(
                xs, a, (((1,), (1,)), ((), ())),
                preferred_element_type=jnp.float32)
            out_ref[0, s] = g
            s += 1


def _grams(xb):
    nb = xb.shape[0]
    return pl.pallas_call(
        _gram_body,
        grid=(nb,),
        in_specs=[pl.BlockSpec((1, 16, _M * _N), lambda b: (b, 0, 0))],
        out_specs=pl.BlockSpec((1, _NS, 16, 16), lambda b: (b, 0, 0, 0)),
        out_shape=jax.ShapeDtypeStruct((nb, _NS, 16, 16), jnp.float32),
    )(xb)


def _make_sc_gather(nb):
    mesh = plsc.VectorSubcoreMesh(core_axis_name="c", subcore_axis_name="s")

    @functools.partial(
        pl.kernel, mesh=mesh,
        out_type=jax.ShapeDtypeStruct((nb * _P_OUT,), jnp.float32),
        scratch_types=[
            pltpu.VMEM((_CPAD,), jnp.int32),
            pltpu.VMEM((_CPAD,), jnp.int32),
            pltpu.VMEM((_CPAD,), jnp.int32),
            pltpu.VMEM((_NS, 16, 16), jnp.float32),
            pltpu.VMEM((_CPAD,), jnp.float32),
        ],
        compiler_params=pltpu.CompilerParams(needs_layout_passes=False),
    )
    def k(g_hbm, is_hbm, ir_hbm, ic_hbm, out_hbm, is_v, ir_v, ic_v, g_v, o_v):
        wid = lax.axis_index("s") * 2 + lax.axis_index("c")
        b = wid // 4
        sub = wid % 4
        pltpu.sync_copy(is_hbm.at[pl.ds(sub * _CPAD, _CPAD)], is_v)
        pltpu.sync_copy(ir_hbm.at[pl.ds(sub * _CPAD, _CPAD)], ir_v)
        pltpu.sync_copy(ic_hbm.at[pl.ds(sub * _CPAD, _CPAD)], ic_v)
        pltpu.sync_copy(g_hbm.at[b], g_v)
        for i in range(_CPAD // 16):
            sl = pl.ds(i * 16, 16)
            o_v[sl] = plsc.load_gather(g_v, [is_v[sl], ir_v[sl], ic_v[sl]])
        off = pl.multiple_of((b * 4 + sub) * _CHUNK, 8)
        pltpu.sync_copy(o_v.at[pl.ds(0, _CHUNK)],
                        out_hbm.at[pl.ds(off, _CHUNK)])

    return k


def kernel(xpsi):
    nb = xpsi.shape[0]
    xb = xpsi.astype(jnp.bfloat16).reshape(nb, 16, _M * _N)
    g = _grams(xb)
    out = _make_sc_gather(nb)(
        g, jnp.asarray(_IDX_S), jnp.asarray(_IDX_R), jnp.asarray(_IDX_C))
    return out.reshape(nb, _P_OUT)


# speedup vs baseline: 1.0058x; 1.0058x over previous
import functools

import numpy as np
import jax
import jax.numpy as jnp
from jax import lax
from jax.experimental import pallas as pl
from jax.experimental.pallas import tpu as pltpu
from jax.experimental.pallas import tpu_sc as plsc

_J = 4
_L = 4
_M = 128
_N = 128


def _build_tables():
    ii = np.arange(_M)
    sx = ((ii + _M // 2) % _M) - _M // 2
    sy = ((np.arange(_N) + _N // 2) % _N) - _N // 2
    SX, SY = np.meshgrid(sx, sy, indexing="ij")
    r = np.sqrt(SX.astype(np.float64) ** 2 + SY.astype(np.float64) ** 2)
    angle_ok = (SX == 0) | (SY == 0) | (SX == SY) | (SX == -SY)
    masks = [(SX == 0) & (SY == 0)]
    for k in range(1, _J + 1):
        masks.append((r <= 2 ** (k - 1)) & angle_ok)
    positions = [np.where(m.reshape(-1))[0] for m in masks]

    la1, la2, kk = [], [], []
    for j1 in range(_J):
        for j2 in range(j1, _J):
            for l1 in range(_L):
                for l2 in range(_L):
                    la1.append(_L * j1 + l1)
                    la2.append(_L * j2 + l2)
                    kk.append(j2 + 1)

    all_shifts = set()
    for q in positions[_J]:
        i, j = divmod(int(q), _N)
        all_shifts.add((int(sx[i]), int(sy[j])))
    groups = {}
    for dx, dy in sorted(all_shifts):
        groups.setdefault(dx, []).append(dy)
    shift_groups = sorted(groups.items(),
                          key=lambda it: ((it[0] % _M) % 8, it[0] % _M))
    sidx_of = {}
    s = 0
    for dx, dys in shift_groups:
        for dy in dys:
            sidx_of[(dx, dy)] = s
            s += 1

    idx = []
    for p in range(len(la1)):
        for q in positions[kk[p]]:
            i, j = divmod(int(q), _N)
            si = sidx_of[(int(sx[i]), int(sy[j]))]
            idx.append((si, la1[p], la2[p]))
    return shift_groups, np.asarray(idx, dtype=np.int32)


_SHIFT_GROUPS, _IDX3 = _build_tables()
_NS = sum(len(d) for _, d in _SHIFT_GROUPS)
_P_OUT = int(_IDX3.shape[0])
_CHUNK = _P_OUT // 4
_CPAD = 1280

_IDX_S = np.zeros((4 * _CPAD,), dtype=np.int32)
_IDX_R = np.zeros((4 * _CPAD,), dtype=np.int32)
_IDX_C = np.zeros((4 * _CPAD,), dtype=np.int32)
for _sub in range(4):
    _part = _IDX3[_sub * _CHUNK:(_sub + 1) * _CHUNK]
    _IDX_S[_sub * _CPAD:_sub * _CPAD + _CHUNK] = _part[:, 0]
    _IDX_R[_sub * _CPAD:_sub * _CPAD + _CHUNK] = _part[:, 1]
    _IDX_C[_sub * _CPAD:_sub * _CPAD + _CHUNK] = _part[:, 2]


def _flat_roll(a, k):
    k %= _M * _N
    if k == 0:
        return a
    return jnp.concatenate([a[:, k:], a[:, :k]], axis=1)


def _gram_body(x_ref, out_ref):
    L = _M * _N
    a = x_ref[0]
    lane = jax.lax.broadcasted_iota(jnp.int32, (16, L), 1) % _N
    dn = (((1,), (1,)), ((), ()))
    rot = {}

    def get_rot(dym):
        if dym not in rot:
            rot[dym] = jnp.where(lane < _N - dym,
                                 _flat_roll(a, dym), _flat_roll(a, dym - _N))
        return rot[dym]

    s = 0
    for dx, dys in _SHIFT_GROUPS:
        k = (_N * dx) % L
        for dy in dys:
            dym = dy % _N
            r = a if dym == 0 else get_rot(dym)
            if k == 0:
                g = lax.dot_general(r, a, dn,
                                    preferred_element_type=jnp.float32)
            else:
                g = lax.dot_general(r[:, k:], a[:, :L - k], dn,
                                    preferred_element_type=jnp.float32)
                g += lax.dot_general(r[:, :k], a[:, L - k:], dn,
                                     preferred_element_type=jnp.float32)
            out_ref[0, s] = g
            s += 1


def _grams(xb):
    nb = xb.shape[0]
    return pl.pallas_call(
        _gram_body,
        grid=(nb,),
        in_specs=[pl.BlockSpec((1, 16, _M * _N), lambda b: (b, 0, 0))],
        out_specs=pl.BlockSpec((1, _NS, 16, 16), lambda b: (b, 0, 0, 0)),
        out_shape=jax.ShapeDtypeStruct((nb, _NS, 16, 16), jnp.float32),
    )(xb)


def _make_sc_gather(nb):
    mesh = plsc.VectorSubcoreMesh(core_axis_name="c", subcore_axis_name="s")

    @functools.partial(
        pl.kernel, mesh=mesh,
        out_type=jax.ShapeDtypeStruct((nb * _P_OUT,), jnp.float32),
        scratch_types=[
            pltpu.VMEM((_CPAD,), jnp.int32),
            pltpu.VMEM((_CPAD,), jnp.int32),
            pltpu.VMEM((_CPAD,), jnp.int32),
            pltpu.VMEM((_NS, 16, 16), jnp.float32),
            pltpu.VMEM((_CPAD,), jnp.float32),
        ],
        compiler_params=pltpu.CompilerParams(needs_layout_passes=False),
    )
    def k(g_hbm, is_hbm, ir_hbm, ic_hbm, out_hbm, is_v, ir_v, ic_v, g_v, o_v):
        wid = lax.axis_index("s") * 2 + lax.axis_index("c")
        b = wid // 4
        sub = wid % 4
        pltpu.sync_copy(is_hbm.at[pl.ds(sub * _CPAD, _CPAD)], is_v)
        pltpu.sync_copy(ir_hbm.at[pl.ds(sub * _CPAD, _CPAD)], ir_v)
        pltpu.sync_copy(ic_hbm.at[pl.ds(sub * _CPAD, _CPAD)], ic_v)
        pltpu.sync_copy(g_hbm.at[b], g_v)
        for i in range(_CPAD // 16):
            sl = pl.ds(i * 16, 16)
            o_v[sl] = plsc.load_gather(g_v, [is_v[sl], ir_v[sl], ic_v[sl]])
        off = pl.multiple_of((b * 4 + sub) * _CHUNK, 8)
        pltpu.sync_copy(o_v.at[pl.ds(0, _CHUNK)],
                        out_hbm.at[pl.ds(off, _CHUNK)])

    return k


def kernel(xpsi):
    nb = xpsi.shape[0]
    xb = xpsi.astype(jnp.bfloat16).reshape(nb, 16, _M * _N)
    g = _grams(xb)
    out = _make_sc_gather(nb)(
        g, jnp.asarray(_IDX_S), jnp.asarray(_IDX_R), jnp.asarray(_IDX_C))
    return out.reshape(nb, _P_OUT)


# speedup vs baseline: 1.8716x; 1.8609x over previous
import functools

import numpy as np
import jax
import jax.numpy as jnp
from jax import lax
from jax.experimental import pallas as pl
from jax.experimental.pallas import tpu as pltpu
from jax.experimental.pallas import tpu_sc as plsc

_J = 4
_L = 4
_M = 128
_N = 128


def _build_tables():
    ii = np.arange(_M)
    sx = ((ii + _M // 2) % _M) - _M // 2
    sy = ((np.arange(_N) + _N // 2) % _N) - _N // 2
    SX, SY = np.meshgrid(sx, sy, indexing="ij")
    r = np.sqrt(SX.astype(np.float64) ** 2 + SY.astype(np.float64) ** 2)
    angle_ok = (SX == 0) | (SY == 0) | (SX == SY) | (SX == -SY)
    masks = [(SX == 0) & (SY == 0)]
    for k in range(1, _J + 1):
        masks.append((r <= 2 ** (k - 1)) & angle_ok)
    positions = [np.where(m.reshape(-1))[0] for m in masks]

    la1, la2, kk = [], [], []
    for j1 in range(_J):
        for j2 in range(j1, _J):
            for l1 in range(_L):
                for l2 in range(_L):
                    la1.append(_L * j1 + l1)
                    la2.append(_L * j2 + l2)
                    kk.append(j2 + 1)

    all_shifts = set()
    for q in positions[_J]:
        i, j = divmod(int(q), _N)
        all_shifts.add((int(sx[i]), int(sy[j])))
    canon = {(dx, dy) for dx, dy in all_shifts
             if dx > 0 or (dx == 0 and dy >= 0)}
    groups = {}
    for dx, dy in sorted(canon):
        groups.setdefault(dx, []).append(dy)
    shift_groups = sorted(groups.items())
    sidx_of = {}
    s = 0
    for dx, dys in shift_groups:
        for dy in dys:
            sidx_of[(dx, dy)] = s
            s += 1

    idx = []
    for p in range(len(la1)):
        for q in positions[kk[p]]:
            i, j = divmod(int(q), _N)
            dx, dy = int(sx[i]), int(sy[j])
            if (dx, dy) in canon:
                idx.append((sidx_of[(dx, dy)], la1[p], la2[p]))
            else:
                idx.append((sidx_of[(-dx, -dy)], la2[p], la1[p]))
    return shift_groups, np.asarray(idx, dtype=np.int32)


_SHIFT_GROUPS, _IDX3 = _build_tables()
_NS = sum(len(d) for _, d in _SHIFT_GROUPS)
_P_OUT = int(_IDX3.shape[0])
_CHUNK = _P_OUT // 4
_CPAD = 1280

_IDX_S = np.zeros((4 * _CPAD,), dtype=np.int32)
_IDX_R = np.zeros((4 * _CPAD,), dtype=np.int32)
_IDX_C = np.zeros((4 * _CPAD,), dtype=np.int32)
for _sub in range(4):
    _part = _IDX3[_sub * _CHUNK:(_sub + 1) * _CHUNK]
    _IDX_S[_sub * _CPAD:_sub * _CPAD + _CHUNK] = _part[:, 0]
    _IDX_R[_sub * _CPAD:_sub * _CPAD + _CHUNK] = _part[:, 1]
    _IDX_C[_sub * _CPAD:_sub * _CPAD + _CHUNK] = _part[:, 2]


def _flat_roll(a, k):
    k %= _M * _N
    if k == 0:
        return a
    return jnp.concatenate([a[:, k:], a[:, :k]], axis=1)


def _gram_body(x_ref, out_ref):
    L = _M * _N
    a = x_ref[...]
    nb = a.shape[0] // 16
    lane = jax.lax.broadcasted_iota(jnp.int32, a.shape, 1) % _N
    dn = (((1,), (1,)), ((), ()))
    rot = {}

    def get_rot(dym):
        if dym not in rot:
            rot[dym] = jnp.where(lane < _N - dym,
                                 _flat_roll(a, dym), _flat_roll(a, dym - _N))
        return rot[dym]

    s = 0
    for dx, dys in _SHIFT_GROUPS:
        k = (_N * dx) % L
        for dy in dys:
            dym = dy % _N
            r = a if dym == 0 else get_rot(dym)
            if k == 0:
                g = lax.dot_general(r, a, dn,
                                    preferred_element_type=jnp.float32)
            else:
                g = lax.dot_general(r[:, k:], a[:, :L - k], dn,
                                    preferred_element_type=jnp.float32)
                g += lax.dot_general(r[:, :k], a[:, L - k:], dn,
                                     preferred_element_type=jnp.float32)
            for b in range(nb):
                out_ref[s, b] = g[16 * b:16 * b + 16, 16 * b:16 * b + 16]
            s += 1


def _grams(xb):
    cc = xb.shape[0]
    nb = cc // 16
    return pl.pallas_call(
        _gram_body,
        in_specs=[pl.BlockSpec((cc, _M * _N), lambda: (0, 0))],
        out_specs=pl.BlockSpec((_NS, nb, 16, 16), lambda: (0, 0, 0, 0)),
        out_shape=jax.ShapeDtypeStruct((_NS, nb, 16, 16), jnp.float32),
    )(xb)


def _make_sc_gather(nb):
    mesh = plsc.VectorSubcoreMesh(core_axis_name="c", subcore_axis_name="s")

    @functools.partial(
        pl.kernel, mesh=mesh,
        out_type=jax.ShapeDtypeStruct((nb * _P_OUT,), jnp.float32),
        scratch_types=[
            pltpu.VMEM((_CPAD,), jnp.int32),
            pltpu.VMEM((_CPAD,), jnp.int32),
            pltpu.VMEM((_CPAD,), jnp.int32),
            pltpu.VMEM((_NS, 16, 16), jnp.float32),
            pltpu.VMEM((_CPAD,), jnp.float32),
        ],
        compiler_params=pltpu.CompilerParams(needs_layout_passes=False),
    )
    def k(g_hbm, is_hbm, ir_hbm, ic_hbm, out_hbm, is_v, ir_v, ic_v, g_v, o_v):
        wid = lax.axis_index("s") * 2 + lax.axis_index("c")
        b = wid // 4
        sub = wid % 4
        pltpu.sync_copy(is_hbm.at[pl.ds(sub * _CPAD, _CPAD)], is_v)
        pltpu.sync_copy(ir_hbm.at[pl.ds(sub * _CPAD, _CPAD)], ir_v)
        pltpu.sync_copy(ic_hbm.at[pl.ds(sub * _CPAD, _CPAD)], ic_v)
        pltpu.sync_copy(g_hbm.at[:, b], g_v)
        for i in range(_CPAD // 16):
            sl = pl.ds(i * 16, 16)
            o_v[sl] = plsc.load_gather(g_v, [is_v[sl], ir_v[sl], ic_v[sl]])
        off = pl.multiple_of((b * 4 + sub) * _CHUNK, 8)
        pltpu.sync_copy(o_v.at[pl.ds(0, _CHUNK)],
                        out_hbm.at[pl.ds(off, _CHUNK)])

    return k


def kernel(xpsi):
    nb = xpsi.shape[0]
    xb = xpsi.astype(jnp.bfloat16).reshape(nb * 16, _M * _N)
    g = _grams(xb)
    out = _make_sc_gather(nb)(
        g, jnp.asarray(_IDX_S), jnp.asarray(_IDX_R), jnp.asarray(_IDX_C))
    return out.reshape(nb, _P_OUT)
